# baseline (speedup 1.0000x reference)
import functools

import jax
import jax.numpy as jnp
from jax import lax
from jax.experimental import pallas as pl
from jax.experimental.pallas import tpu as pltpu
from jax.experimental.pallas import tpu_sc as plsc

N = 10000
D = 128
E = 320000
NPAD = 10240
NC = 2
NS = 16
NW = NC * NS
EW = E // NW
RPT = NPAD // NS

@functools.lru_cache(maxsize=None)
def _mesh():
    return plsc.VectorSubcoreMesh(core_axis_name="c", subcore_axis_name="s")


@functools.lru_cache(maxsize=None)
def _make_deg_kernel():
    chunk, k = 125, 80

    @functools.partial(
        pl.kernel,
        out_type=jax.ShapeDtypeStruct((NC, NPAD, 16), jnp.float32),
        scratch_types=[
            pltpu.VMEM((k, chunk), jnp.int32),
            pltpu.VMEM((chunk, 16), jnp.float32),
            pltpu.VMEM_SHARED((NPAD, 16), jnp.float32),
            pltpu.SemaphoreType.DMA,
            pltpu.SemaphoreType.DMA,
        ],
        mesh=_mesh(),
        compiler_params=pltpu.CompilerParams(use_tc_tiling_on_sc=False),
    )
    def deg_kernel(ei_hbm, ones_hbm, zeros_hbm, out_hbm, dst_v, ones_v, acc,
                   sem0, sem1):
        c = lax.axis_index("c")
        s = lax.axis_index("s")
        wid = s * NC + c
        pltpu.sync_copy(ei_hbm.at[1, wid], dst_v)
        pltpu.sync_copy(ones_hbm, ones_v)
        pltpu.sync_copy(zeros_hbm, acc.at[pl.ds(s * RPT, RPT)])
        plsc.subcore_barrier()
        sems = (sem0, sem1)

        pltpu.async_copy(ones_v, acc.at[dst_v.at[0]], sem0, add=True)
        pltpu.async_copy(ones_v, acc.at[dst_v.at[1]], sem1, add=True)

        def outer(g, carry):
            for bb in range(2):
                j = 2 + 2 * g + bb
                pltpu.make_async_copy(ones_v, acc.at[dst_v.at[j - 2]],
                                      sems[bb]).wait()
                pltpu.async_copy(ones_v, acc.at[dst_v.at[j]], sems[bb],
                                 add=True)
            return carry

        lax.fori_loop(0, (k - 2) // 2, outer, 0)
        pltpu.make_async_copy(ones_v, acc.at[dst_v.at[k - 2]], sem0).wait()
        pltpu.make_async_copy(ones_v, acc.at[dst_v.at[k - 1]], sem1).wait()
        plsc.subcore_barrier()
        pltpu.sync_copy(acc.at[pl.ds(s * RPT, RPT)],
                        out_hbm.at[c, pl.ds(s * RPT, RPT)])

    return deg_kernel


@functools.lru_cache(maxsize=None)
def _make_agg_kernel(w, chunk, nbuf):
    k = EW // chunk

    @functools.partial(
        pl.kernel,
        out_type=jax.ShapeDtypeStruct((NC, NPAD, w), jnp.float32),
        scratch_types=[
            pltpu.VMEM((k, chunk), jnp.int32),
            pltpu.VMEM((k, chunk), jnp.int32),
        ] + [pltpu.VMEM((chunk, w), jnp.float32) for _ in range(nbuf)]
          + [pltpu.VMEM_SHARED((NPAD, w), jnp.float32)]
          + [pltpu.SemaphoreType.DMA for _ in range(nbuf)],
        mesh=_mesh(),
        compiler_params=pltpu.CompilerParams(use_tc_tiling_on_sc=False),
    )
    def agg_kernel(ei_hbm, g_hbm, zeros_hbm, out_hbm, src_v, dst_v, *rest):
        bufs = rest[:nbuf]
        acc = rest[nbuf]
        sems = rest[nbuf + 1:]
        c = lax.axis_index("c")
        s = lax.axis_index("s")
        wid = s * NC + c
        pltpu.sync_copy(ei_hbm.at[0, wid], src_v)
        pltpu.sync_copy(ei_hbm.at[1, wid], dst_v)
        pltpu.sync_copy(zeros_hbm, acc.at[pl.ds(s * RPT, RPT)])
        plsc.subcore_barrier()

        for b in range(nbuf):
            pltpu.async_copy(g_hbm.at[src_v.at[b]], bufs[b], sems[b])

        def outer(g, carry):
            for b in range(nbuf):
                j = g * nbuf + b
                pltpu.make_async_copy(g_hbm.at[src_v.at[j]], bufs[b],
                                      sems[b]).wait()
                pltpu.sync_copy(bufs[b], acc.at[dst_v.at[j]], add=True)
                pltpu.async_copy(g_hbm.at[src_v.at[j + nbuf]], bufs[b],
                                 sems[b])
            return carry

        lax.fori_loop(0, k // nbuf - 1, outer, 0)
        for b in range(nbuf):
            j = k - nbuf + b
            pltpu.make_async_copy(g_hbm.at[src_v.at[j]], bufs[b],
                                  sems[b]).wait()
            pltpu.sync_copy(bufs[b], acc.at[dst_v.at[j]], add=True)

        plsc.subcore_barrier()
        pltpu.sync_copy(acc.at[pl.ds(s * RPT, RPT)],
                        out_hbm.at[c, pl.ds(s * RPT, RPT)])

    return agg_kernel


_B = 1000
_GRID = N // _B


def _dinv_from_parts(degp):
    d0 = degp[0, :, 0:1]
    d1 = degp[1, :, 0:1]
    return lax.rsqrt(d0 + d1 + 1.0)


def _tc_a_body(x_ref, w_ref, degp_ref, g_ref):
    dinv = _dinv_from_parts(degp_ref[...])
    h = jnp.dot(x_ref[...], w_ref[...], preferred_element_type=jnp.float32)
    g_ref[...] = h * dinv


def _tc_b_body(aggp_ref, g_ref, degp_ref, b1_ref, w2_ref, gp_ref):
    dinv = _dinv_from_parts(degp_ref[...])
    a = aggp_ref[0] + aggp_ref[1] + g_ref[...]
    out1 = jnp.maximum(dinv * a + b1_ref[...], 0.0)
    p = jnp.dot(out1, w2_ref[...], preferred_element_type=jnp.float32)
    gp_ref[...] = p * dinv


def _tc_c_body(aggp_ref, gp_ref, degp_ref, b2_ref, out_ref):
    dinv = _dinv_from_parts(degp_ref[...])
    a = aggp_ref[0] + aggp_ref[1] + gp_ref[...]
    z = dinv * a + b2_ref[...]
    out_ref[...] = 1.0 / (1.0 + jnp.exp(-z))


def _row_spec(width):
    return pl.BlockSpec((_B, width), lambda i: (i, 0))


def _part_spec(width):
    return pl.BlockSpec((NC, _B, width), lambda i: (0, i, 0))


_tc_a = pl.pallas_call(
    _tc_a_body,
    grid=(_GRID,),
    in_specs=[_row_spec(D),
              pl.BlockSpec((D, D), lambda i: (0, 0)),
              _part_spec(16)],
    out_specs=_row_spec(D),
    out_shape=jax.ShapeDtypeStruct((N, D), jnp.float32),
)

_tc_b = pl.pallas_call(
    _tc_b_body,
    grid=(_GRID,),
    in_specs=[_part_spec(D),
              _row_spec(D),
              _part_spec(16),
              pl.BlockSpec((1, D), lambda i: (0, 0)),
              pl.BlockSpec((D, 16), lambda i: (0, 0))],
    out_specs=_row_spec(16),
    out_shape=jax.ShapeDtypeStruct((N, 16), jnp.float32),
)

_tc_c = pl.pallas_call(
    _tc_c_body,
    grid=(_GRID,),
    in_specs=[_part_spec(16),
              _row_spec(16),
              _part_spec(16),
              pl.BlockSpec((1, 16), lambda i: (0, 0))],
    out_specs=_row_spec(16),
    out_shape=jax.ShapeDtypeStruct((N, 16), jnp.float32),
)


def kernel(x, edge_index, W1, b1, W2, b2):
    ei = edge_index.astype(jnp.int32)
    ei_125 = ei.reshape(2, NW, 80, 125)
    ei_50 = ei.reshape(2, NW, 200, 50)
    ei_100 = ei.reshape(2, NW, 100, 100)
    ones16 = jnp.ones((125, 16), jnp.float32)
    zeros16 = jnp.zeros((RPT, 16), jnp.float32)
    zeros128 = jnp.zeros((RPT, D), jnp.float32)

    degp = _make_deg_kernel()(ei_125, ones16, zeros16)
    g1 = _tc_a(x, W1, degp)
    agg1p = _make_agg_kernel(D, 50, 4)(ei_50, g1, zeros128)

    w2p = jnp.zeros((D, 16), jnp.float32).at[:, :2].set(W2)
    b1r = b1.reshape(1, D)
    gp = _tc_b(agg1p, g1, degp, b1r, w2p)

    agg2p = _make_agg_kernel(16, 100, 4)(ei_100, gp, zeros16)
    b2p = jnp.zeros((1, 16), jnp.float32).at[0, :2].set(b2)
    out = _tc_c(agg2p, gp, degp, b2p)
    return out[:, :2]


# speedup vs baseline: 1.0251x; 1.0251x over previous
import functools

import jax
import jax.numpy as jnp
from jax import lax
from jax.experimental import pallas as pl
from jax.experimental.pallas import tpu as pltpu
from jax.experimental.pallas import tpu_sc as plsc

N = 10000
D = 128
E = 320000
NPAD = 10240
NC = 2
NS = 16
NW = NC * NS
EW = E // NW
RPT = NPAD // NS

@functools.lru_cache(maxsize=None)
def _mesh():
    return plsc.VectorSubcoreMesh(core_axis_name="c", subcore_axis_name="s")


@functools.lru_cache(maxsize=None)
def _make_deg_kernel():
    chunk, k = 125, 80

    @functools.partial(
        pl.kernel,
        out_type=jax.ShapeDtypeStruct((NC, NPAD, 16), jnp.float32),
        scratch_types=[
            pltpu.VMEM((k, chunk), jnp.int32),
            pltpu.VMEM((chunk, 16), jnp.float32),
            pltpu.VMEM_SHARED((NPAD, 16), jnp.float32),
            pltpu.SemaphoreType.DMA,
            pltpu.SemaphoreType.DMA,
        ],
        mesh=_mesh(),
        compiler_params=pltpu.CompilerParams(use_tc_tiling_on_sc=False),
    )
    def deg_kernel(ei_hbm, ones_hbm, zeros_hbm, out_hbm, dst_v, ones_v, acc,
                   sem0, sem1):
        c = lax.axis_index("c")
        s = lax.axis_index("s")
        wid = s * NC + c
        pltpu.sync_copy(ei_hbm.at[1, wid], dst_v)
        pltpu.sync_copy(ones_hbm, ones_v)
        pltpu.sync_copy(zeros_hbm, acc.at[pl.ds(s * RPT, RPT)])
        plsc.subcore_barrier()
        sems = (sem0, sem1)

        pltpu.async_copy(ones_v, acc.at[dst_v.at[0]], sem0, add=True)
        pltpu.async_copy(ones_v, acc.at[dst_v.at[1]], sem1, add=True)

        def outer(g, carry):
            for bb in range(2):
                j = 2 + 2 * g + bb
                pltpu.make_async_copy(ones_v, acc.at[dst_v.at[j - 2]],
                                      sems[bb]).wait()
                pltpu.async_copy(ones_v, acc.at[dst_v.at[j]], sems[bb],
                                 add=True)
            return carry

        lax.fori_loop(0, (k - 2) // 2, outer, 0)
        pltpu.make_async_copy(ones_v, acc.at[dst_v.at[k - 2]], sem0).wait()
        pltpu.make_async_copy(ones_v, acc.at[dst_v.at[k - 1]], sem1).wait()
        plsc.subcore_barrier()
        pltpu.sync_copy(acc.at[pl.ds(s * RPT, RPT)],
                        out_hbm.at[c, pl.ds(s * RPT, RPT)])

    return deg_kernel


@functools.lru_cache(maxsize=None)
def _make_agg_kernel(w, chunk, nbuf):
    k = EW // chunk

    @functools.partial(
        pl.kernel,
        out_type=jax.ShapeDtypeStruct((NC, NPAD, w), jnp.float32),
        scratch_types=[
            pltpu.VMEM((k, chunk), jnp.int32),
            pltpu.VMEM((k, chunk), jnp.int32),
        ] + [pltpu.VMEM((chunk, w), jnp.float32) for _ in range(nbuf)]
          + [pltpu.VMEM_SHARED((NPAD, w), jnp.float32)]
          + [pltpu.SemaphoreType.DMA for _ in range(nbuf)],
        mesh=_mesh(),
        compiler_params=pltpu.CompilerParams(use_tc_tiling_on_sc=False),
    )
    def agg_kernel(ei_hbm, g_hbm, zeros_hbm, out_hbm, src_v, dst_v, *rest):
        bufs = rest[:nbuf]
        acc = rest[nbuf]
        sems = rest[nbuf + 1:]
        c = lax.axis_index("c")
        s = lax.axis_index("s")
        wid = s * NC + c
        pltpu.sync_copy(ei_hbm.at[0, wid], src_v)
        pltpu.sync_copy(ei_hbm.at[1, wid], dst_v)
        pltpu.sync_copy(zeros_hbm, acc.at[pl.ds(s * RPT, RPT)])
        plsc.subcore_barrier()

        for b in range(nbuf):
            pltpu.async_copy(g_hbm.at[src_v.at[b]], bufs[b], sems[b])

        def outer(g, carry):
            for b in range(nbuf):
                j = g * nbuf + b
                pltpu.make_async_copy(g_hbm.at[src_v.at[j]], bufs[b],
                                      sems[b]).wait()
                pltpu.sync_copy(bufs[b], acc.at[dst_v.at[j]], add=True)
                pltpu.async_copy(g_hbm.at[src_v.at[j + nbuf]], bufs[b],
                                 sems[b])
            return carry

        lax.fori_loop(0, k // nbuf - 1, outer, 0)
        for b in range(nbuf):
            j = k - nbuf + b
            pltpu.make_async_copy(g_hbm.at[src_v.at[j]], bufs[b],
                                  sems[b]).wait()
            pltpu.sync_copy(bufs[b], acc.at[dst_v.at[j]], add=True)

        plsc.subcore_barrier()
        pltpu.sync_copy(acc.at[pl.ds(s * RPT, RPT)],
                        out_hbm.at[c, pl.ds(s * RPT, RPT)])

    return agg_kernel


_B = 2000
_GRID = N // _B


def _dinv_from_parts(degp):
    d0 = degp[0, :, 0:1]
    d1 = degp[1, :, 0:1]
    return lax.rsqrt(d0 + d1 + 1.0)


def _tc_a_body(x_ref, w_ref, degp_ref, g_ref):
    dinv = _dinv_from_parts(degp_ref[...])
    h = jnp.dot(x_ref[...], w_ref[...], preferred_element_type=jnp.float32)
    g_ref[...] = h * dinv


def _tc_b_body(aggp_ref, g_ref, degp_ref, b1_ref, w2_ref, gp_ref):
    dinv = _dinv_from_parts(degp_ref[...])
    a = aggp_ref[0] + aggp_ref[1] + g_ref[...]
    out1 = jnp.maximum(dinv * a + b1_ref[...], 0.0)
    p = jnp.dot(out1, w2_ref[...], preferred_element_type=jnp.float32)
    gp_ref[...] = p * dinv


def _tc_c_body(aggp_ref, gp_ref, degp_ref, b2_ref, out_ref):
    dinv = _dinv_from_parts(degp_ref[...])
    a = aggp_ref[0] + aggp_ref[1] + gp_ref[...]
    z = dinv * a + b2_ref[...]
    out_ref[...] = 1.0 / (1.0 + jnp.exp(-z))


def _row_spec(width):
    return pl.BlockSpec((_B, width), lambda i: (i, 0))


def _part_spec(width):
    return pl.BlockSpec((NC, _B, width), lambda i: (0, i, 0))


_tc_a = pl.pallas_call(
    _tc_a_body,
    grid=(_GRID,),
    in_specs=[_row_spec(D),
              pl.BlockSpec((D, D), lambda i: (0, 0)),
              _part_spec(16)],
    out_specs=_row_spec(D),
    out_shape=jax.ShapeDtypeStruct((N, D), jnp.float32),
)

_tc_b = pl.pallas_call(
    _tc_b_body,
    grid=(_GRID,),
    in_specs=[_part_spec(D),
              _row_spec(D),
              _part_spec(16),
              pl.BlockSpec((1, D), lambda i: (0, 0)),
              pl.BlockSpec((D, 16), lambda i: (0, 0))],
    out_specs=_row_spec(16),
    out_shape=jax.ShapeDtypeStruct((N, 16), jnp.float32),
)

_tc_c = pl.pallas_call(
    _tc_c_body,
    grid=(_GRID,),
    in_specs=[_part_spec(16),
              _row_spec(16),
              _part_spec(16),
              pl.BlockSpec((1, 16), lambda i: (0, 0))],
    out_specs=_row_spec(16),
    out_shape=jax.ShapeDtypeStruct((N, 16), jnp.float32),
)


def kernel(x, edge_index, W1, b1, W2, b2):
    ei = edge_index.astype(jnp.int32)
    ei_125 = ei.reshape(2, NW, 80, 125)
    ei_50 = ei.reshape(2, NW, 200, 50)
    ei_100 = ei.reshape(2, NW, 100, 100)
    ones16 = jnp.ones((125, 16), jnp.float32)
    zeros16 = jnp.zeros((RPT, 16), jnp.float32)
    zeros128 = jnp.zeros((RPT, D), jnp.float32)

    degp = _make_deg_kernel()(ei_125, ones16, zeros16)
    g1 = _tc_a(x, W1, degp)
    agg1p = _make_agg_kernel(D, 50, 4)(ei_50, g1, zeros128)

    w2p = jnp.zeros((D, 16), jnp.float32).at[:, :2].set(W2)
    b1r = b1.reshape(1, D)
    gp = _tc_b(agg1p, g1, degp, b1r, w2p)

    agg2p = _make_agg_kernel(16, 100, 4)(ei_100, gp, zeros16)
    b2p = jnp.zeros((1, 16), jnp.float32).at[0, :2].set(b2)
    out = _tc_c(agg2p, gp, degp, b2p)
    return out[:, :2]


# speedup vs baseline: 1.0513x; 1.0256x over previous
import functools

import jax
import jax.numpy as jnp
from jax import lax
from jax.experimental import pallas as pl
from jax.experimental.pallas import tpu as pltpu
from jax.experimental.pallas import tpu_sc as plsc

N = 10000
D = 128
E = 320000
NPAD = 10240
NC = 2
NS = 16
NW = NC * NS
EW = E // NW
RPT = NPAD // NS

@functools.lru_cache(maxsize=None)
def _mesh():
    return plsc.VectorSubcoreMesh(core_axis_name="c", subcore_axis_name="s")


@functools.lru_cache(maxsize=None)
def _make_deg_kernel():
    chunk, k = 125, 80

    @functools.partial(
        pl.kernel,
        out_type=jax.ShapeDtypeStruct((NC, NPAD, 16), jnp.float32),
        scratch_types=[
            pltpu.VMEM((k, chunk), jnp.int32),
            pltpu.VMEM((chunk, 16), jnp.float32),
            pltpu.VMEM_SHARED((NPAD, 16), jnp.float32),
            pltpu.SemaphoreType.DMA,
            pltpu.SemaphoreType.DMA,
        ],
        mesh=_mesh(),
        compiler_params=pltpu.CompilerParams(use_tc_tiling_on_sc=False),
    )
    def deg_kernel(ei_hbm, ones_hbm, zeros_hbm, out_hbm, dst_v, ones_v, acc,
                   sem0, sem1):
        c = lax.axis_index("c")
        s = lax.axis_index("s")
        wid = s * NC + c
        pltpu.sync_copy(ei_hbm.at[1, wid], dst_v)
        pltpu.sync_copy(ones_hbm, ones_v)
        pltpu.sync_copy(zeros_hbm, acc.at[pl.ds(s * RPT, RPT)])
        plsc.subcore_barrier()
        sems = (sem0, sem1)

        pltpu.async_copy(ones_v, acc.at[dst_v.at[0]], sem0, add=True)
        pltpu.async_copy(ones_v, acc.at[dst_v.at[1]], sem1, add=True)

        def outer(g, carry):
            for bb in range(2):
                j = 2 + 2 * g + bb
                pltpu.make_async_copy(ones_v, acc.at[dst_v.at[j - 2]],
                                      sems[bb]).wait()
                pltpu.async_copy(ones_v, acc.at[dst_v.at[j]], sems[bb],
                                 add=True)
            return carry

        lax.fori_loop(0, (k - 2) // 2, outer, 0)
        pltpu.make_async_copy(ones_v, acc.at[dst_v.at[k - 2]], sem0).wait()
        pltpu.make_async_copy(ones_v, acc.at[dst_v.at[k - 1]], sem1).wait()
        plsc.subcore_barrier()
        pltpu.sync_copy(acc.at[pl.ds(s * RPT, RPT)],
                        out_hbm.at[c, pl.ds(s * RPT, RPT)])

    return deg_kernel


@functools.lru_cache(maxsize=None)
def _make_agg_kernel(w, chunk, nbuf, stage_src=False):
    k = EW // chunk

    @functools.partial(
        pl.kernel,
        out_type=jax.ShapeDtypeStruct((NC, NPAD, w), jnp.float32),
        scratch_types=[
            pltpu.VMEM((k, chunk), jnp.int32),
            pltpu.VMEM((k, chunk), jnp.int32),
        ] + [pltpu.VMEM((chunk, w), jnp.float32) for _ in range(nbuf)]
          + [pltpu.VMEM_SHARED((NPAD, w), jnp.float32)]
          + ([pltpu.VMEM_SHARED((N, w), jnp.float32)] if stage_src else [])
          + [pltpu.SemaphoreType.DMA for _ in range(nbuf)],
        mesh=_mesh(),
        compiler_params=pltpu.CompilerParams(use_tc_tiling_on_sc=False),
    )
    def agg_kernel(ei_hbm, g_hbm, zeros_hbm, out_hbm, src_v, dst_v, *rest):
        bufs = rest[:nbuf]
        acc = rest[nbuf]
        if stage_src:
            gsrc = rest[nbuf + 1]
            sems = rest[nbuf + 2:]
        else:
            gsrc = g_hbm
            sems = rest[nbuf + 1:]
        c = lax.axis_index("c")
        s = lax.axis_index("s")
        wid = s * NC + c
        pltpu.sync_copy(ei_hbm.at[0, wid], src_v)
        pltpu.sync_copy(ei_hbm.at[1, wid], dst_v)
        pltpu.sync_copy(zeros_hbm, acc.at[pl.ds(s * RPT, RPT)])
        if stage_src:
            pltpu.sync_copy(g_hbm.at[pl.ds(s * (N // NS), N // NS)],
                            gsrc.at[pl.ds(s * (N // NS), N // NS)])
        plsc.subcore_barrier()

        for b in range(nbuf):
            pltpu.async_copy(gsrc.at[src_v.at[b]], bufs[b], sems[b])

        def outer(g, carry):
            for b in range(nbuf):
                j = g * nbuf + b
                pltpu.make_async_copy(gsrc.at[src_v.at[j]], bufs[b],
                                      sems[b]).wait()
                pltpu.sync_copy(bufs[b], acc.at[dst_v.at[j]], add=True)
                pltpu.async_copy(gsrc.at[src_v.at[j + nbuf]], bufs[b],
                                 sems[b])
            return carry

        lax.fori_loop(0, k // nbuf - 1, outer, 0)
        for b in range(nbuf):
            j = k - nbuf + b
            pltpu.make_async_copy(gsrc.at[src_v.at[j]], bufs[b],
                                  sems[b]).wait()
            pltpu.sync_copy(bufs[b], acc.at[dst_v.at[j]], add=True)

        plsc.subcore_barrier()
        pltpu.sync_copy(acc.at[pl.ds(s * RPT, RPT)],
                        out_hbm.at[c, pl.ds(s * RPT, RPT)])

    return agg_kernel


_B = 2000
_GRID = N // _B


def _dinv_from_parts(degp):
    d0 = degp[0, :, 0:1]
    d1 = degp[1, :, 0:1]
    return lax.rsqrt(d0 + d1 + 1.0)


def _tc_a_body(x_ref, w_ref, degp_ref, g_ref):
    dinv = _dinv_from_parts(degp_ref[...])
    h = jnp.dot(x_ref[...], w_ref[...], preferred_element_type=jnp.float32)
    g_ref[...] = h * dinv


def _tc_b_body(aggp_ref, g_ref, degp_ref, b1_ref, w2_ref, gp_ref):
    dinv = _dinv_from_parts(degp_ref[...])
    a = aggp_ref[0] + aggp_ref[1] + g_ref[...]
    out1 = jnp.maximum(dinv * a + b1_ref[...], 0.0)
    p = jnp.dot(out1, w2_ref[...], preferred_element_type=jnp.float32)
    gp_ref[...] = p * dinv


def _tc_c_body(aggp_ref, gp_ref, degp_ref, b2_ref, out_ref):
    dinv = _dinv_from_parts(degp_ref[...])
    a = aggp_ref[0] + aggp_ref[1] + gp_ref[...]
    z = dinv * a + b2_ref[...]
    out_ref[...] = 1.0 / (1.0 + jnp.exp(-z))


def _row_spec(width):
    return pl.BlockSpec((_B, width), lambda i: (i, 0))


def _part_spec(width):
    return pl.BlockSpec((NC, _B, width), lambda i: (0, i, 0))


_tc_a = pl.pallas_call(
    _tc_a_body,
    grid=(_GRID,),
    in_specs=[_row_spec(D),
              pl.BlockSpec((D, D), lambda i: (0, 0)),
              _part_spec(16)],
    out_specs=_row_spec(D),
    out_shape=jax.ShapeDtypeStruct((N, D), jnp.float32),
)

_tc_b = pl.pallas_call(
    _tc_b_body,
    grid=(_GRID,),
    in_specs=[_part_spec(D),
              _row_spec(D),
              _part_spec(16),
              pl.BlockSpec((1, D), lambda i: (0, 0)),
              pl.BlockSpec((D, 16), lambda i: (0, 0))],
    out_specs=_row_spec(16),
    out_shape=jax.ShapeDtypeStruct((N, 16), jnp.float32),
)

_tc_c = pl.pallas_call(
    _tc_c_body,
    grid=(_GRID,),
    in_specs=[_part_spec(16),
              _row_spec(16),
              _part_spec(16),
              pl.BlockSpec((1, 16), lambda i: (0, 0))],
    out_specs=_row_spec(16),
    out_shape=jax.ShapeDtypeStruct((N, 16), jnp.float32),
)


def kernel(x, edge_index, W1, b1, W2, b2):
    ei = edge_index.astype(jnp.int32)
    ei_125 = ei.reshape(2, NW, 80, 125)
    ei_50 = ei.reshape(2, NW, 200, 50)
    ei_100 = ei.reshape(2, NW, 100, 100)
    ones16 = jnp.ones((125, 16), jnp.float32)
    zeros16 = jnp.zeros((RPT, 16), jnp.float32)
    zeros128 = jnp.zeros((RPT, D), jnp.float32)

    degp = _make_deg_kernel()(ei_125, ones16, zeros16)
    g1 = _tc_a(x, W1, degp)
    agg1p = _make_agg_kernel(D, 50, 4)(ei_50, g1, zeros128)

    w2p = jnp.zeros((D, 16), jnp.float32).at[:, :2].set(W2)
    b1r = b1.reshape(1, D)
    gp = _tc_b(agg1p, g1, degp, b1r, w2p)

    agg2p = _make_agg_kernel(16, 100, 4, stage_src=True)(ei_100, gp, zeros16)
    b2p = jnp.zeros((1, 16), jnp.float32).at[0, :2].set(b2)
    out = _tc_c(agg2p, gp, degp, b2p)
    return out[:, :2]
